# baseline (speedup 1.0000x reference)
import functools

import jax
import jax.numpy as jnp
from jax import lax
from jax.experimental import pallas as pl
from jax.experimental.pallas import tpu as pltpu
from jax.experimental.pallas import tpu_sc as plsc

N_VAR = 5000
N_CLAUSE = 5000
E = 160000
D = 128
B_GRAPHS = 8
STEP = 4

NP = 5120
CHUNK = 128
NW = 32
NJ = 40
E_PAD = NW * NJ * CHUNK


def _embed_body(emb_ref, x_ref, o_ref):
    e0 = emb_ref[0:1, :]
    e1 = emb_ref[1:2, :]
    xf = x_ref[...].astype(jnp.float32)
    o_ref[...] = e0 + xf * (e1 - e0)


def _embed(emb, x_col):
    return pl.pallas_call(
        _embed_body,
        out_shape=jax.ShapeDtypeStruct((NP, D), jnp.float32),
    )(emb, x_col)


def _msg_body(x_ref, w0_ref, b0_ref, w1_ref, b1_ref, w2_ref, b2_ref, o_ref):
    x = x_ref[...]
    x = jnp.maximum(jnp.dot(x, w0_ref[...], preferred_element_type=jnp.float32)
                    + b0_ref[...], 0.0)
    x = jnp.maximum(jnp.dot(x, w1_ref[...], preferred_element_type=jnp.float32)
                    + b1_ref[...], 0.0)
    x = jnp.maximum(jnp.dot(x, w2_ref[...], preferred_element_type=jnp.float32)
                    + b2_ref[...], 0.0)
    o_ref[...] = x


def _msg(x, w0t, b0, w1t, b1, w2t, b2):
    return pl.pallas_call(
        _msg_body,
        out_shape=jax.ShapeDtypeStruct((NP, D), jnp.float32),
    )(x, w0t, b0, w1t, b1, w2t, b2)


def _lstm_body(part_ref, h_ref, c_ref, wih_ref, whh_ref, b_ref,
               ho_ref, co_ref):
    agg = part_ref[0:NP, :] + part_ref[NP:2 * NP, :]
    h0 = h_ref[...]
    c0 = c_ref[...]
    g = (jnp.dot(agg, wih_ref[...], preferred_element_type=jnp.float32)
         + jnp.dot(h0, whh_ref[...], preferred_element_type=jnp.float32)
         + b_ref[...])
    ig = jax.nn.sigmoid(g[:, 0 * D:1 * D])
    fg = jax.nn.sigmoid(g[:, 1 * D:2 * D])
    gg = jnp.tanh(g[:, 2 * D:3 * D])
    og = jax.nn.sigmoid(g[:, 3 * D:4 * D])
    c2 = fg * c0 + ig * gg
    ho_ref[...] = og * jnp.tanh(c2)
    co_ref[...] = jnp.maximum(c2, 0.0)


def _lstm(part, h0, c0, wih_t, whh_t, b):
    return pl.pallas_call(
        _lstm_body,
        out_shape=[jax.ShapeDtypeStruct((NP, D), jnp.float32),
                   jax.ShapeDtypeStruct((NP, D), jnp.float32)],
    )(part, h0, c0, wih_t, whh_t, b)


def _pool_body(x_ref, gid_ref, gw_ref, gb_ref, w0_ref, b0_ref, w1_ref, b1_ref,
               w2_ref, b2_ref, o_ref):
    x = x_ref[...]
    gw = gw_ref[...]
    gate = jnp.sum(x * gw, axis=1, keepdims=True) + gb_ref[0, 0]
    gid = gid_ref[...]
    giota = lax.broadcasted_iota(jnp.int32, (x.shape[0], B_GRAPHS), 1)
    maskt = gid == giota
    neg = jnp.float32(-1e30)
    gm = jnp.where(maskt, gate, neg)
    gmax = jnp.max(gm, axis=0, keepdims=True)
    e8 = jnp.where(maskt, jnp.exp(gm - gmax), 0.0)
    den = jnp.sum(e8, axis=0, keepdims=True)
    w8 = e8 / den
    ro = jax.lax.dot_general(w8, x, (((0,), (0,)), ((), ())),
                             preferred_element_type=jnp.float32)
    y = jnp.maximum(jnp.dot(ro, w0_ref[...], preferred_element_type=jnp.float32)
                    + b0_ref[...], 0.0)
    y = jnp.maximum(jnp.dot(y, w1_ref[...], preferred_element_type=jnp.float32)
                    + b1_ref[...], 0.0)
    y = jnp.dot(y, w2_ref[...], preferred_element_type=jnp.float32) + b2_ref[...]
    o_ref[...] = y


def _pool(x, gid_col, gw, gb, w0t, b0, w1t, b1, w2t_pad, b2_pad):
    return pl.pallas_call(
        _pool_body,
        out_shape=jax.ShapeDtypeStruct((B_GRAPHS, D), jnp.float32),
    )(x, gid_col, gw, gb, w0t, b0, w1t, b1, w2t_pad, b2_pad)


_SC_MESH = plsc.VectorSubcoreMesh(core_axis_name="c", subcore_axis_name="s")


@functools.partial(
    pl.kernel, mesh=_SC_MESH,
    out_type=jax.ShapeDtypeStruct((2 * NP, D), jnp.float32),
    scratch_types=(
        [pltpu.VMEM((CHUNK,), jnp.int32)] * 16
        + [pltpu.VMEM((CHUNK, D), jnp.float32)] * 2
        + [pltpu.VMEM_SHARED((NP, D), jnp.float32)] * 2
        + [pltpu.SemaphoreType.DMA] * 10
    ),
)
def _segsum_sc(m_hbm, src_hbm, dst_hbm, z_hbm, out_hbm, *scr):
    sidx = scr[0:16:2]
    didx = scr[1:16:2]
    rows = scr[16:18]
    m_sh, acc_sh = scr[18], scr[19]
    semi = scr[20:28]
    semg = scr[28:30]
    cid = lax.axis_index("c")
    sid = lax.axis_index("s")
    wid = sid * 2 + cid
    rpt = NP // 16
    pltpu.sync_copy(z_hbm.at[pl.ds(sid * rpt, rpt)],
                    acc_sh.at[pl.ds(sid * rpt, rpt)])
    plsc.subcore_barrier()

    UNROLL = 8

    def body(i, carry):
        base = (wid * NJ + UNROLL * i) * CHUNK
        hs = []
        for k in range(UNROLL):
            hs.append(pltpu.async_copy(
                src_hbm.at[pl.ds(base + k * CHUNK, CHUNK)], sidx[k], semi[k]))
            hs.append(pltpu.async_copy(
                dst_hbm.at[pl.ds(base + k * CHUNK, CHUNK)], didx[k], semi[k]))
        gs = [None] * UNROLL
        for k in range(2):
            hs[2 * k].wait()
            hs[2 * k + 1].wait()
            gs[k] = pltpu.async_copy(m_hbm.at[sidx[k]], rows[k], semg[k])
        for k in range(UNROLL):
            gs[k].wait()
            pltpu.sync_copy(rows[k % 2], acc_sh.at[didx[k]], add=True)
            if k + 2 < UNROLL:
                hs[2 * (k + 2)].wait()
                hs[2 * (k + 2) + 1].wait()
                gs[k + 2] = pltpu.async_copy(m_hbm.at[sidx[k + 2]],
                                             rows[k % 2], semg[k % 2])
        return carry

    lax.fori_loop(0, NJ // UNROLL, body, 0)
    plsc.subcore_barrier()
    pltpu.sync_copy(acc_sh.at[pl.ds(sid * rpt, rpt)],
                    out_hbm.at[pl.ds(cid * NP + sid * rpt, rpt)])


def _segsum(m, src, dst, zeros):
    return _segsum_sc(m, src, dst, zeros)


def kernel(params, var_x, clause_x, edge_v2c, edge_c2v, clause_graph_id):
    p = params
    row = lambda v: v.reshape(1, -1)

    def pad_nodes(x, val):
        return jnp.concatenate(
            [x.astype(jnp.int32), jnp.full((NP - x.shape[0],), val,
                                           jnp.int32)]).reshape(-1, 1)

    ev = _embed(p["embed"], pad_nodes(var_x, 0))
    ec = _embed(p["embed"], pad_nodes(clause_x, 0))

    h_v = c_v = ev
    h_c = c_c = ec

    wt = {}
    for et in ("v2c", "c2v"):
        for j in range(3):
            wt[et + str(j)] = p[et + "_W" + str(j)].T
    lw = {}
    for li in range(2):
        lw[li] = (p["lstm%d_Wih" % li].T, p["lstm%d_Whh" % li].T,
                  row(p["lstm%d_bih" % li] + p["lstm%d_bhh" % li]))

    def pad_src(s):
        return jnp.concatenate([s.astype(jnp.int32),
                                jnp.zeros((E_PAD - E,), jnp.int32)])

    def pad_dst(d):
        return jnp.concatenate([d.astype(jnp.int32),
                                jnp.full((E_PAD - E,), NP - 1, jnp.int32)])

    src_v2c = pad_src(edge_v2c[0])
    dst_v2c = pad_dst(edge_v2c[1])
    src_c2v = pad_src(edge_c2v[0])
    dst_c2v = pad_dst(edge_c2v[1])

    zeros = jnp.zeros((NP, D), jnp.float32)

    for _ in range(STEP):
        m = _msg(c_v, wt["v2c0"], row(p["v2c_b0"]), wt["v2c1"],
                 row(p["v2c_b1"]), wt["v2c2"], row(p["v2c_b2"]))
        part = _segsum(m, src_v2c, dst_v2c, zeros)
        h_c, c_c = _lstm(part, h_c, c_c, *lw[0])

        m = _msg(c_c, wt["c2v0"], row(p["c2v_b0"]), wt["c2v1"],
                 row(p["c2v_b1"]), wt["c2v2"], row(p["c2v_b2"]))
        part = _segsum(m, src_c2v, dst_c2v, zeros)
        h_v, c_v = _lstm(part, h_v, c_v, *lw[1])

    w2t_pad = jnp.zeros((D, D), jnp.float32).at[:, :2].set(p["mlp_W2"].T)
    b2_pad = jnp.zeros((1, D), jnp.float32).at[0, :2].set(p["mlp_b2"])
    y_pad = _pool(c_c, pad_nodes(clause_graph_id, B_GRAPHS),
                  row(p["gate_W"][0]), p["gate_b"].reshape(1, 1),
                  p["mlp_W0"].T, row(p["mlp_b0"]),
                  p["mlp_W1"].T, row(p["mlp_b1"]), w2t_pad, b2_pad)
    return y_pad[:, :2]


# speedup vs baseline: 2.7144x; 2.7144x over previous
import functools

import jax
import jax.numpy as jnp
from jax import lax
from jax.experimental import pallas as pl
from jax.experimental.pallas import tpu as pltpu
from jax.experimental.pallas import tpu_sc as plsc

N_VAR = 5000
N_CLAUSE = 5000
E = 160000
D = 128
B_GRAPHS = 8
STEP = 4

NP = 5120
CHUNK = 128
NW = 32
NJ = 40
E_PAD = NW * NJ * CHUNK


def _embed_body(emb_ref, x_ref, o_ref):
    e0 = emb_ref[0:1, :]
    e1 = emb_ref[1:2, :]
    xf = x_ref[...].astype(jnp.float32)
    o_ref[...] = e0 + xf * (e1 - e0)


def _embed(emb, x_col):
    return pl.pallas_call(
        _embed_body,
        out_shape=jax.ShapeDtypeStruct((NP, D), jnp.float32),
    )(emb, x_col)


def _msg_body(x_ref, w0_ref, b0_ref, w1_ref, b1_ref, w2_ref, b2_ref, o_ref):
    x = x_ref[...]
    x = jnp.maximum(jnp.dot(x, w0_ref[...], preferred_element_type=jnp.float32)
                    + b0_ref[...], 0.0)
    x = jnp.maximum(jnp.dot(x, w1_ref[...], preferred_element_type=jnp.float32)
                    + b1_ref[...], 0.0)
    x = jnp.maximum(jnp.dot(x, w2_ref[...], preferred_element_type=jnp.float32)
                    + b2_ref[...], 0.0)
    o_ref[...] = x


def _msg(x, w0t, b0, w1t, b1, w2t, b2):
    return pl.pallas_call(
        _msg_body,
        out_shape=jax.ShapeDtypeStruct((NP, D), jnp.float32),
    )(x, w0t, b0, w1t, b1, w2t, b2)


def _lstm_body(part_ref, h_ref, c_ref, wih_ref, whh_ref, b_ref,
               ho_ref, co_ref):
    agg = part_ref[0:NP, :] + part_ref[NP:2 * NP, :]
    h0 = h_ref[...]
    c0 = c_ref[...]
    g = (jnp.dot(agg, wih_ref[...], preferred_element_type=jnp.float32)
         + jnp.dot(h0, whh_ref[...], preferred_element_type=jnp.float32)
         + b_ref[...])
    ig = jax.nn.sigmoid(g[:, 0 * D:1 * D])
    fg = jax.nn.sigmoid(g[:, 1 * D:2 * D])
    gg = jnp.tanh(g[:, 2 * D:3 * D])
    og = jax.nn.sigmoid(g[:, 3 * D:4 * D])
    c2 = fg * c0 + ig * gg
    ho_ref[...] = og * jnp.tanh(c2)
    co_ref[...] = jnp.maximum(c2, 0.0)


def _lstm(part, h0, c0, wih_t, whh_t, b):
    return pl.pallas_call(
        _lstm_body,
        out_shape=[jax.ShapeDtypeStruct((NP, D), jnp.float32),
                   jax.ShapeDtypeStruct((NP, D), jnp.float32)],
    )(part, h0, c0, wih_t, whh_t, b)


def _pool_body(x_ref, gid_ref, gw_ref, gb_ref, w0_ref, b0_ref, w1_ref, b1_ref,
               w2_ref, b2_ref, o_ref):
    x = x_ref[...]
    gw = gw_ref[...]
    gate = jnp.sum(x * gw, axis=1, keepdims=True) + gb_ref[0, 0]
    gid = gid_ref[...]
    giota = lax.broadcasted_iota(jnp.int32, (x.shape[0], B_GRAPHS), 1)
    maskt = gid == giota
    neg = jnp.float32(-1e30)
    gm = jnp.where(maskt, gate, neg)
    gmax = jnp.max(gm, axis=0, keepdims=True)
    e8 = jnp.where(maskt, jnp.exp(gm - gmax), 0.0)
    den = jnp.sum(e8, axis=0, keepdims=True)
    w8 = e8 / den
    ro = jax.lax.dot_general(w8, x, (((0,), (0,)), ((), ())),
                             preferred_element_type=jnp.float32)
    y = jnp.maximum(jnp.dot(ro, w0_ref[...], preferred_element_type=jnp.float32)
                    + b0_ref[...], 0.0)
    y = jnp.maximum(jnp.dot(y, w1_ref[...], preferred_element_type=jnp.float32)
                    + b1_ref[...], 0.0)
    y = jnp.dot(y, w2_ref[...], preferred_element_type=jnp.float32) + b2_ref[...]
    o_ref[...] = y


def _pool(x, gid_col, gw, gb, w0t, b0, w1t, b1, w2t_pad, b2_pad):
    return pl.pallas_call(
        _pool_body,
        out_shape=jax.ShapeDtypeStruct((B_GRAPHS, D), jnp.float32),
    )(x, gid_col, gw, gb, w0t, b0, w1t, b1, w2t_pad, b2_pad)


_SC_MESH = plsc.VectorSubcoreMesh(core_axis_name="c", subcore_axis_name="s")


@functools.partial(
    pl.kernel, mesh=_SC_MESH,
    out_type=jax.ShapeDtypeStruct((2 * NP, D), jnp.float32),
    scratch_types=(
        [pltpu.VMEM((CHUNK,), jnp.int32)] * 8
        + [pltpu.VMEM((CHUNK, D), jnp.float32)] * 2
        + [pltpu.VMEM_SHARED((NP, D), jnp.float32)] * 2
        + [pltpu.SemaphoreType.DMA] * 6
    ),
)
def _segsum_sc(m_hbm, src_hbm, dst_hbm, z_hbm, out_hbm, *scr):
    sidx = scr[0:8:2]
    didx = scr[1:8:2]
    rows = scr[8:10]
    m_sh, acc_sh = scr[10], scr[11]
    semi = scr[12:16]
    semg = scr[16:18]
    cid = lax.axis_index("c")
    sid = lax.axis_index("s")
    wid = sid * 2 + cid
    rpt = NP // 16
    hz = pltpu.async_copy(z_hbm.at[pl.ds(sid * rpt, rpt)],
                          acc_sh.at[pl.ds(sid * rpt, rpt)], semi[0])
    hm = pltpu.async_copy(m_hbm.at[pl.ds(sid * rpt, rpt)],
                          m_sh.at[pl.ds(sid * rpt, rpt)], semi[1])
    hz.wait()
    hm.wait()
    plsc.subcore_barrier()

    def body(i, carry):
        base = (wid * NJ + 2 * i) * CHUNK
        h1 = pltpu.async_copy(src_hbm.at[pl.ds(base, CHUNK)],
                              sidx[0], semi[0])
        h2 = pltpu.async_copy(dst_hbm.at[pl.ds(base, CHUNK)],
                              didx[0], semi[0])
        h3 = pltpu.async_copy(src_hbm.at[pl.ds(base + CHUNK, CHUNK)],
                              sidx[1], semi[1])
        h4 = pltpu.async_copy(dst_hbm.at[pl.ds(base + CHUNK, CHUNK)],
                              didx[1], semi[1])
        h1.wait()
        h2.wait()
        g0 = pltpu.async_copy(m_sh.at[sidx[0]], rows[0], semg[0])
        h3.wait()
        h4.wait()
        g1 = pltpu.async_copy(m_sh.at[sidx[1]], rows[1], semg[1])
        g0.wait()
        pltpu.sync_copy(rows[0], acc_sh.at[didx[0]], add=True)
        g1.wait()
        pltpu.sync_copy(rows[1], acc_sh.at[didx[1]], add=True)
        return carry

    lax.fori_loop(0, NJ // 2, body, 0)
    plsc.subcore_barrier()
    pltpu.sync_copy(acc_sh.at[pl.ds(sid * rpt, rpt)],
                    out_hbm.at[pl.ds(cid * NP + sid * rpt, rpt)])


def _segsum(m, src, dst, zeros):
    return _segsum_sc(m, src, dst, zeros)


def kernel(params, var_x, clause_x, edge_v2c, edge_c2v, clause_graph_id):
    p = params
    row = lambda v: v.reshape(1, -1)

    def pad_nodes(x, val):
        return jnp.concatenate(
            [x.astype(jnp.int32), jnp.full((NP - x.shape[0],), val,
                                           jnp.int32)]).reshape(-1, 1)

    ev = _embed(p["embed"], pad_nodes(var_x, 0))
    ec = _embed(p["embed"], pad_nodes(clause_x, 0))

    h_v = c_v = ev
    h_c = c_c = ec

    wt = {}
    for et in ("v2c", "c2v"):
        for j in range(3):
            wt[et + str(j)] = p[et + "_W" + str(j)].T
    lw = {}
    for li in range(2):
        lw[li] = (p["lstm%d_Wih" % li].T, p["lstm%d_Whh" % li].T,
                  row(p["lstm%d_bih" % li] + p["lstm%d_bhh" % li]))

    def pad_src(s):
        return jnp.concatenate([s.astype(jnp.int32),
                                jnp.zeros((E_PAD - E,), jnp.int32)])

    def pad_dst(d):
        return jnp.concatenate([d.astype(jnp.int32),
                                jnp.full((E_PAD - E,), NP - 1, jnp.int32)])

    src_v2c = pad_src(edge_v2c[0])
    dst_v2c = pad_dst(edge_v2c[1])
    src_c2v = pad_src(edge_c2v[0])
    dst_c2v = pad_dst(edge_c2v[1])

    zeros = jnp.zeros((NP, D), jnp.float32)

    for _ in range(STEP):
        m = _msg(c_v, wt["v2c0"], row(p["v2c_b0"]), wt["v2c1"],
                 row(p["v2c_b1"]), wt["v2c2"], row(p["v2c_b2"]))
        part = _segsum(m, src_v2c, dst_v2c, zeros)
        h_c, c_c = _lstm(part, h_c, c_c, *lw[0])

        m = _msg(c_c, wt["c2v0"], row(p["c2v_b0"]), wt["c2v1"],
                 row(p["c2v_b1"]), wt["c2v2"], row(p["c2v_b2"]))
        part = _segsum(m, src_c2v, dst_c2v, zeros)
        h_v, c_v = _lstm(part, h_v, c_v, *lw[1])

    w2t_pad = jnp.zeros((D, D), jnp.float32).at[:, :2].set(p["mlp_W2"].T)
    b2_pad = jnp.zeros((1, D), jnp.float32).at[0, :2].set(p["mlp_b2"])
    y_pad = _pool(c_c, pad_nodes(clause_graph_id, B_GRAPHS),
                  row(p["gate_W"][0]), p["gate_b"].reshape(1, 1),
                  p["mlp_W0"].T, row(p["mlp_b0"]),
                  p["mlp_W1"].T, row(p["mlp_b1"]), w2t_pad, b2_pad)
    return y_pad[:, :2]


# speedup vs baseline: 2.7148x; 1.0002x over previous
import functools

import jax
import jax.numpy as jnp
from jax import lax
from jax.experimental import pallas as pl
from jax.experimental.pallas import tpu as pltpu
from jax.experimental.pallas import tpu_sc as plsc

N_VAR = 5000
N_CLAUSE = 5000
E = 160000
D = 128
B_GRAPHS = 8
STEP = 4

NP = 5120
CHUNK = 128
NW = 32
NJ = 40
E_PAD = NW * NJ * CHUNK


def _embed_body(emb_ref, x_ref, o_ref):
    e0 = emb_ref[0:1, :]
    e1 = emb_ref[1:2, :]
    xf = x_ref[...].astype(jnp.float32)
    o_ref[...] = e0 + xf * (e1 - e0)


def _embed(emb, x_col):
    return pl.pallas_call(
        _embed_body,
        out_shape=jax.ShapeDtypeStruct((NP, D), jnp.float32),
    )(emb, x_col)


def _msg_body(x_ref, w0_ref, b0_ref, w1_ref, b1_ref, w2_ref, b2_ref, o_ref):
    x = x_ref[...]
    x = jnp.maximum(jnp.dot(x, w0_ref[...], preferred_element_type=jnp.float32)
                    + b0_ref[...], 0.0)
    x = jnp.maximum(jnp.dot(x, w1_ref[...], preferred_element_type=jnp.float32)
                    + b1_ref[...], 0.0)
    x = jnp.maximum(jnp.dot(x, w2_ref[...], preferred_element_type=jnp.float32)
                    + b2_ref[...], 0.0)
    o_ref[...] = x


def _msg(x, w0t, b0, w1t, b1, w2t, b2):
    return pl.pallas_call(
        _msg_body,
        out_shape=jax.ShapeDtypeStruct((NP, D), jnp.float32),
    )(x, w0t, b0, w1t, b1, w2t, b2)


def _lstm_body(part_ref, h_ref, c_ref, wih_ref, whh_ref, b_ref,
               ho_ref, co_ref):
    agg = part_ref[0:NP, :] + part_ref[NP:2 * NP, :]
    h0 = h_ref[...]
    c0 = c_ref[...]
    g = (jnp.dot(agg, wih_ref[...], preferred_element_type=jnp.float32)
         + jnp.dot(h0, whh_ref[...], preferred_element_type=jnp.float32)
         + b_ref[...])
    ig = jax.nn.sigmoid(g[:, 0 * D:1 * D])
    fg = jax.nn.sigmoid(g[:, 1 * D:2 * D])
    gg = jnp.tanh(g[:, 2 * D:3 * D])
    og = jax.nn.sigmoid(g[:, 3 * D:4 * D])
    c2 = fg * c0 + ig * gg
    ho_ref[...] = og * jnp.tanh(c2)
    co_ref[...] = jnp.maximum(c2, 0.0)


def _lstm(part, h0, c0, wih_t, whh_t, b):
    return pl.pallas_call(
        _lstm_body,
        out_shape=[jax.ShapeDtypeStruct((NP, D), jnp.float32),
                   jax.ShapeDtypeStruct((NP, D), jnp.float32)],
    )(part, h0, c0, wih_t, whh_t, b)


def _pool_body(x_ref, gid_ref, gw_ref, gb_ref, w0_ref, b0_ref, w1_ref, b1_ref,
               w2_ref, b2_ref, o_ref):
    x = x_ref[...]
    gw = gw_ref[...]
    gate = jnp.sum(x * gw, axis=1, keepdims=True) + gb_ref[0, 0]
    gid = gid_ref[...]
    giota = lax.broadcasted_iota(jnp.int32, (x.shape[0], B_GRAPHS), 1)
    maskt = gid == giota
    neg = jnp.float32(-1e30)
    gm = jnp.where(maskt, gate, neg)
    gmax = jnp.max(gm, axis=0, keepdims=True)
    e8 = jnp.where(maskt, jnp.exp(gm - gmax), 0.0)
    den = jnp.sum(e8, axis=0, keepdims=True)
    w8 = e8 / den
    ro = jax.lax.dot_general(w8, x, (((0,), (0,)), ((), ())),
                             preferred_element_type=jnp.float32)
    y = jnp.maximum(jnp.dot(ro, w0_ref[...], preferred_element_type=jnp.float32)
                    + b0_ref[...], 0.0)
    y = jnp.maximum(jnp.dot(y, w1_ref[...], preferred_element_type=jnp.float32)
                    + b1_ref[...], 0.0)
    y = jnp.dot(y, w2_ref[...], preferred_element_type=jnp.float32) + b2_ref[...]
    o_ref[...] = y


def _pool(x, gid_col, gw, gb, w0t, b0, w1t, b1, w2t_pad, b2_pad):
    return pl.pallas_call(
        _pool_body,
        out_shape=jax.ShapeDtypeStruct((B_GRAPHS, D), jnp.float32),
    )(x, gid_col, gw, gb, w0t, b0, w1t, b1, w2t_pad, b2_pad)


_SC_MESH = plsc.VectorSubcoreMesh(core_axis_name="c", subcore_axis_name="s")


@functools.partial(
    pl.kernel, mesh=_SC_MESH,
    out_type=jax.ShapeDtypeStruct((2 * NP, D), jnp.float32),
    scratch_types=(
        [pltpu.VMEM((CHUNK,), jnp.int32)] * 8
        + [pltpu.VMEM((CHUNK, D), jnp.float32)] * 2
        + [pltpu.VMEM_SHARED((NP, D), jnp.float32)] * 2
        + [pltpu.SemaphoreType.DMA] * 6
    ),
)
def _segsum_sc(m_hbm, src_hbm, dst_hbm, z_hbm, out_hbm, *scr):
    sidx = scr[0:8:2]
    didx = scr[1:8:2]
    rows = scr[8:10]
    m_sh, acc_sh = scr[10], scr[11]
    semi = scr[12:16]
    semg = scr[16:18]
    cid = lax.axis_index("c")
    sid = lax.axis_index("s")
    wid = sid * 2 + cid
    rpt = NP // 16
    hz = pltpu.async_copy(z_hbm.at[pl.ds(sid * rpt, rpt)],
                          acc_sh.at[pl.ds(sid * rpt, rpt)], semi[0])
    hm = pltpu.async_copy(m_hbm.at[pl.ds(sid * rpt, rpt)],
                          m_sh.at[pl.ds(sid * rpt, rpt)], semi[1])
    hz.wait()
    hm.wait()
    plsc.subcore_barrier()

    def body(i, carry):
        base = (wid * NJ + 2 * i) * CHUNK
        h1 = pltpu.async_copy(src_hbm.at[pl.ds(base, CHUNK)],
                              sidx[0], semi[0])
        h2 = pltpu.async_copy(dst_hbm.at[pl.ds(base, CHUNK)],
                              didx[0], semi[0])
        h3 = pltpu.async_copy(src_hbm.at[pl.ds(base + CHUNK, CHUNK)],
                              sidx[1], semi[1])
        h4 = pltpu.async_copy(dst_hbm.at[pl.ds(base + CHUNK, CHUNK)],
                              didx[1], semi[1])
        h1.wait()
        h2.wait()
        g0 = pltpu.async_copy(m_sh.at[sidx[0]], rows[0], semg[0])
        h3.wait()
        h4.wait()
        g1 = pltpu.async_copy(m_sh.at[sidx[1]], rows[1], semg[1])
        g0.wait()
        pltpu.sync_copy(rows[0], acc_sh.at[didx[0]], add=True)
        g1.wait()
        pltpu.sync_copy(rows[1], acc_sh.at[didx[1]], add=True)
        return carry

    lax.fori_loop(0, NJ // 2, body, 0)
    plsc.subcore_barrier()
    pltpu.sync_copy(acc_sh.at[pl.ds(sid * rpt, rpt)],
                    out_hbm.at[pl.ds(cid * NP + sid * rpt, rpt)])


def _segsum(m, src, dst, zeros):
    return _segsum_sc(m, src, dst, zeros)


def kernel(params, var_x, clause_x, edge_v2c, edge_c2v, clause_graph_id):
    p = params
    row = lambda v: v.reshape(1, -1)

    def pad_nodes(x, val):
        return jnp.concatenate(
            [x.astype(jnp.int32), jnp.full((NP - x.shape[0],), val,
                                           jnp.int32)]).reshape(-1, 1)

    ev = _embed(p["embed"], pad_nodes(var_x, 0))
    ec = _embed(p["embed"], pad_nodes(clause_x, 0))

    h_v = c_v = ev
    h_c = c_c = ec

    wt = {}
    for et in ("v2c", "c2v"):
        for j in range(3):
            wt[et + str(j)] = p[et + "_W" + str(j)].T
    lw = {}
    for li in range(2):
        lw[li] = (p["lstm%d_Wih" % li].T, p["lstm%d_Whh" % li].T,
                  row(p["lstm%d_bih" % li] + p["lstm%d_bhh" % li]))

    def pad_src(s):
        return jnp.concatenate([s.astype(jnp.int32),
                                jnp.zeros((E_PAD - E,), jnp.int32)])

    def pad_dst(d):
        return jnp.concatenate([d.astype(jnp.int32),
                                jnp.full((E_PAD - E,), NP - 1, jnp.int32)])

    src_v2c = pad_src(edge_v2c[0])
    dst_v2c = pad_dst(edge_v2c[1])
    src_c2v = pad_src(edge_c2v[0])
    dst_c2v = pad_dst(edge_c2v[1])

    zeros = jnp.zeros((NP, D), jnp.float32)

    for step in range(STEP):
        m = _msg(c_v, wt["v2c0"], row(p["v2c_b0"]), wt["v2c1"],
                 row(p["v2c_b1"]), wt["v2c2"], row(p["v2c_b2"]))
        part = _segsum(m, src_v2c, dst_v2c, zeros)
        h_c, c_c = _lstm(part, h_c, c_c, *lw[0])

        if step == STEP - 1:
            break
        m = _msg(c_c, wt["c2v0"], row(p["c2v_b0"]), wt["c2v1"],
                 row(p["c2v_b1"]), wt["c2v2"], row(p["c2v_b2"]))
        part = _segsum(m, src_c2v, dst_c2v, zeros)
        h_v, c_v = _lstm(part, h_v, c_v, *lw[1])

    w2t_pad = jnp.zeros((D, D), jnp.float32).at[:, :2].set(p["mlp_W2"].T)
    b2_pad = jnp.zeros((1, D), jnp.float32).at[0, :2].set(p["mlp_b2"])
    y_pad = _pool(c_c, pad_nodes(clause_graph_id, B_GRAPHS),
                  row(p["gate_W"][0]), p["gate_b"].reshape(1, 1),
                  p["mlp_W0"].T, row(p["mlp_b0"]),
                  p["mlp_W1"].T, row(p["mlp_b1"]), w2t_pad, b2_pad)
    return y_pad[:, :2]


# speedup vs baseline: 2.8063x; 1.0337x over previous
import functools

import jax
import jax.numpy as jnp
from jax import lax
from jax.experimental import pallas as pl
from jax.experimental.pallas import tpu as pltpu
from jax.experimental.pallas import tpu_sc as plsc

N_VAR = 5000
N_CLAUSE = 5000
E = 160000
D = 128
B_GRAPHS = 8
STEP = 4

NP = 5120
CHUNK = 128
NW = 32
NJ = 40
E_PAD = NW * NJ * CHUNK


def _embed_body(emb_ref, x_ref, o_ref):
    e0 = emb_ref[0:1, :]
    e1 = emb_ref[1:2, :]
    xf = x_ref[...].astype(jnp.float32)
    o_ref[...] = e0 + xf * (e1 - e0)


def _embed(emb, x_col):
    return pl.pallas_call(
        _embed_body,
        out_shape=jax.ShapeDtypeStruct((NP, D), jnp.float32),
    )(emb, x_col)


def _msg_body(x_ref, w0_ref, b0_ref, w1_ref, b1_ref, w2_ref, b2_ref, o_ref):
    x = x_ref[...]
    x = jnp.maximum(jnp.dot(x, w0_ref[...], preferred_element_type=jnp.float32)
                    + b0_ref[...], 0.0)
    x = jnp.maximum(jnp.dot(x, w1_ref[...], preferred_element_type=jnp.float32)
                    + b1_ref[...], 0.0)
    x = jnp.maximum(jnp.dot(x, w2_ref[...], preferred_element_type=jnp.float32)
                    + b2_ref[...], 0.0)
    o_ref[...] = x


def _msg(x, w0t, b0, w1t, b1, w2t, b2):
    return pl.pallas_call(
        _msg_body,
        out_shape=jax.ShapeDtypeStruct((NP, D), jnp.float32),
    )(x, w0t, b0, w1t, b1, w2t, b2)


def _lstm_body(part_ref, h_ref, c_ref, wih_ref, whh_ref, b_ref,
               ho_ref, co_ref):
    agg = part_ref[0:NP, :] + part_ref[NP:2 * NP, :]
    h0 = h_ref[...]
    c0 = c_ref[...]
    g = (jnp.dot(agg, wih_ref[...], preferred_element_type=jnp.float32)
         + jnp.dot(h0, whh_ref[...], preferred_element_type=jnp.float32)
         + b_ref[...])
    ig = jax.nn.sigmoid(g[:, 0 * D:1 * D])
    fg = jax.nn.sigmoid(g[:, 1 * D:2 * D])
    gg = jnp.tanh(g[:, 2 * D:3 * D])
    og = jax.nn.sigmoid(g[:, 3 * D:4 * D])
    c2 = fg * c0 + ig * gg
    ho_ref[...] = og * jnp.tanh(c2)
    co_ref[...] = jnp.maximum(c2, 0.0)


def _lstm(part, h0, c0, wih_t, whh_t, b):
    return pl.pallas_call(
        _lstm_body,
        out_shape=[jax.ShapeDtypeStruct((NP, D), jnp.float32),
                   jax.ShapeDtypeStruct((NP, D), jnp.float32)],
    )(part, h0, c0, wih_t, whh_t, b)


def _lstm_msg_body(part_ref, h_ref, c_ref, wih_ref, whh_ref, b_ref,
                   w0_ref, b0_ref, w1_ref, b1_ref, w2_ref, b2_ref,
                   ho_ref, co_ref, mo_ref):
    agg = part_ref[0:NP, :] + part_ref[NP:2 * NP, :]
    h0 = h_ref[...]
    c0 = c_ref[...]
    g = (jnp.dot(agg, wih_ref[...], preferred_element_type=jnp.float32)
         + jnp.dot(h0, whh_ref[...], preferred_element_type=jnp.float32)
         + b_ref[...])
    ig = jax.nn.sigmoid(g[:, 0 * D:1 * D])
    fg = jax.nn.sigmoid(g[:, 1 * D:2 * D])
    gg = jnp.tanh(g[:, 2 * D:3 * D])
    og = jax.nn.sigmoid(g[:, 3 * D:4 * D])
    c2 = fg * c0 + ig * gg
    ho_ref[...] = og * jnp.tanh(c2)
    c = jnp.maximum(c2, 0.0)
    co_ref[...] = c
    x = jnp.maximum(jnp.dot(c, w0_ref[...], preferred_element_type=jnp.float32)
                    + b0_ref[...], 0.0)
    x = jnp.maximum(jnp.dot(x, w1_ref[...], preferred_element_type=jnp.float32)
                    + b1_ref[...], 0.0)
    x = jnp.maximum(jnp.dot(x, w2_ref[...], preferred_element_type=jnp.float32)
                    + b2_ref[...], 0.0)
    mo_ref[...] = x


def _lstm_msg(part, h0, c0, lstm_w, msg_w):
    return pl.pallas_call(
        _lstm_msg_body,
        out_shape=[jax.ShapeDtypeStruct((NP, D), jnp.float32),
                   jax.ShapeDtypeStruct((NP, D), jnp.float32),
                   jax.ShapeDtypeStruct((NP, D), jnp.float32)],
    )(part, h0, c0, *lstm_w, *msg_w)


def _pool_body(x_ref, gid_ref, gw_ref, gb_ref, w0_ref, b0_ref, w1_ref, b1_ref,
               w2_ref, b2_ref, o_ref):
    x = x_ref[...]
    gw = gw_ref[...]
    gate = jnp.sum(x * gw, axis=1, keepdims=True) + gb_ref[0, 0]
    gid = gid_ref[...]
    giota = lax.broadcasted_iota(jnp.int32, (x.shape[0], B_GRAPHS), 1)
    maskt = gid == giota
    neg = jnp.float32(-1e30)
    gm = jnp.where(maskt, gate, neg)
    gmax = jnp.max(gm, axis=0, keepdims=True)
    e8 = jnp.where(maskt, jnp.exp(gm - gmax), 0.0)
    den = jnp.sum(e8, axis=0, keepdims=True)
    w8 = e8 / den
    ro = jax.lax.dot_general(w8, x, (((0,), (0,)), ((), ())),
                             preferred_element_type=jnp.float32)
    y = jnp.maximum(jnp.dot(ro, w0_ref[...], preferred_element_type=jnp.float32)
                    + b0_ref[...], 0.0)
    y = jnp.maximum(jnp.dot(y, w1_ref[...], preferred_element_type=jnp.float32)
                    + b1_ref[...], 0.0)
    y = jnp.dot(y, w2_ref[...], preferred_element_type=jnp.float32) + b2_ref[...]
    o_ref[...] = y


def _pool(x, gid_col, gw, gb, w0t, b0, w1t, b1, w2t_pad, b2_pad):
    return pl.pallas_call(
        _pool_body,
        out_shape=jax.ShapeDtypeStruct((B_GRAPHS, D), jnp.float32),
    )(x, gid_col, gw, gb, w0t, b0, w1t, b1, w2t_pad, b2_pad)


_SC_MESH = plsc.VectorSubcoreMesh(core_axis_name="c", subcore_axis_name="s")


@functools.partial(
    pl.kernel, mesh=_SC_MESH,
    out_type=jax.ShapeDtypeStruct((2 * NP, D), jnp.float32),
    scratch_types=(
        [pltpu.VMEM((CHUNK,), jnp.int32)] * 8
        + [pltpu.VMEM((CHUNK, D), jnp.float32)] * 2
        + [pltpu.VMEM_SHARED((NP, D), jnp.float32)] * 2
        + [pltpu.SemaphoreType.DMA] * 6
    ),
)
def _segsum_sc(m_hbm, src_hbm, dst_hbm, z_hbm, out_hbm, *scr):
    sidx = scr[0:8:2]
    didx = scr[1:8:2]
    rows = scr[8:10]
    m_sh, acc_sh = scr[10], scr[11]
    semi = scr[12:16]
    semg = scr[16:18]
    cid = lax.axis_index("c")
    sid = lax.axis_index("s")
    wid = sid * 2 + cid
    rpt = NP // 16
    hz = pltpu.async_copy(z_hbm.at[pl.ds(sid * rpt, rpt)],
                          acc_sh.at[pl.ds(sid * rpt, rpt)], semi[0])
    hm = pltpu.async_copy(m_hbm.at[pl.ds(sid * rpt, rpt)],
                          m_sh.at[pl.ds(sid * rpt, rpt)], semi[1])
    hz.wait()
    hm.wait()
    plsc.subcore_barrier()

    def body(i, carry):
        base = (wid * NJ + 2 * i) * CHUNK
        h1 = pltpu.async_copy(src_hbm.at[pl.ds(base, CHUNK)],
                              sidx[0], semi[0])
        h2 = pltpu.async_copy(dst_hbm.at[pl.ds(base, CHUNK)],
                              didx[0], semi[0])
        h3 = pltpu.async_copy(src_hbm.at[pl.ds(base + CHUNK, CHUNK)],
                              sidx[1], semi[1])
        h4 = pltpu.async_copy(dst_hbm.at[pl.ds(base + CHUNK, CHUNK)],
                              didx[1], semi[1])
        h1.wait()
        h2.wait()
        g0 = pltpu.async_copy(m_sh.at[sidx[0]], rows[0], semg[0])
        h3.wait()
        h4.wait()
        g1 = pltpu.async_copy(m_sh.at[sidx[1]], rows[1], semg[1])
        g0.wait()
        pltpu.sync_copy(rows[0], acc_sh.at[didx[0]], add=True)
        g1.wait()
        pltpu.sync_copy(rows[1], acc_sh.at[didx[1]], add=True)
        return carry

    lax.fori_loop(0, NJ // 2, body, 0)
    plsc.subcore_barrier()
    pltpu.sync_copy(acc_sh.at[pl.ds(sid * rpt, rpt)],
                    out_hbm.at[pl.ds(cid * NP + sid * rpt, rpt)])


def _segsum(m, src, dst, zeros):
    return _segsum_sc(m, src, dst, zeros)


def kernel(params, var_x, clause_x, edge_v2c, edge_c2v, clause_graph_id):
    p = params
    row = lambda v: v.reshape(1, -1)

    def pad_nodes(x, val):
        return jnp.concatenate(
            [x.astype(jnp.int32), jnp.full((NP - x.shape[0],), val,
                                           jnp.int32)]).reshape(-1, 1)

    ev = _embed(p["embed"], pad_nodes(var_x, 0))
    ec = _embed(p["embed"], pad_nodes(clause_x, 0))

    h_v = c_v = ev
    h_c = c_c = ec

    wt = {}
    for et in ("v2c", "c2v"):
        for j in range(3):
            wt[et + str(j)] = p[et + "_W" + str(j)].T
    lw = {}
    for li in range(2):
        lw[li] = (p["lstm%d_Wih" % li].T, p["lstm%d_Whh" % li].T,
                  row(p["lstm%d_bih" % li] + p["lstm%d_bhh" % li]))

    def pad_src(s):
        return jnp.concatenate([s.astype(jnp.int32),
                                jnp.zeros((E_PAD - E,), jnp.int32)])

    def pad_dst(d):
        return jnp.concatenate([d.astype(jnp.int32),
                                jnp.full((E_PAD - E,), NP - 1, jnp.int32)])

    src_v2c = pad_src(edge_v2c[0])
    dst_v2c = pad_dst(edge_v2c[1])
    src_c2v = pad_src(edge_c2v[0])
    dst_c2v = pad_dst(edge_c2v[1])

    zeros = jnp.zeros((NP, D), jnp.float32)

    mw = {et: (wt[et + "0"], row(p[et + "_b0"]), wt[et + "1"],
               row(p[et + "_b1"]), wt[et + "2"], row(p[et + "_b2"]))
          for et in ("v2c", "c2v")}

    m = _msg(c_v, *mw["v2c"])
    for step in range(STEP):
        part = _segsum(m, src_v2c, dst_v2c, zeros)
        if step == STEP - 1:
            h_c, c_c = _lstm(part, h_c, c_c, *lw[0])
            break
        h_c, c_c, m = _lstm_msg(part, h_c, c_c, lw[0], mw["c2v"])
        part = _segsum(m, src_c2v, dst_c2v, zeros)
        h_v, c_v, m = _lstm_msg(part, h_v, c_v, lw[1], mw["v2c"])

    w2t_pad = jnp.zeros((D, D), jnp.float32).at[:, :2].set(p["mlp_W2"].T)
    b2_pad = jnp.zeros((1, D), jnp.float32).at[0, :2].set(p["mlp_b2"])
    y_pad = _pool(c_c, pad_nodes(clause_graph_id, B_GRAPHS),
                  row(p["gate_W"][0]), p["gate_b"].reshape(1, 1),
                  p["mlp_W0"].T, row(p["mlp_b0"]),
                  p["mlp_W1"].T, row(p["mlp_b1"]), w2t_pad, b2_pad)
    return y_pad[:, :2]
